# baseline (speedup 1.0000x reference)
import functools

import jax
import jax.numpy as jnp
from jax import lax
from jax.experimental import pallas as pl
from jax.experimental.pallas import tpu as pltpu
from jax.experimental.pallas import tpu_sc as plsc

EPS = 1e-12
L = 16


def kernel(P_vocab, attn_dist, p_gen, target_idx, copy_idx):
    B, V = P_vocab.shape
    S = attn_dist.shape[1]

    pvT = P_vocab.T
    atT = attn_dist.T
    pgT = p_gen.T
    tg = target_idx.astype(jnp.int32)
    cp = copy_idx.astype(jnp.int32)

    info = plsc.get_sparse_core_info()
    NC, NS = info.num_cores, info.num_subcores
    NW = NC * NS
    b_per_w = B // NW
    n_chunks = b_per_w // L

    mesh = plsc.VectorSubcoreMesh(core_axis_name="c", subcore_axis_name="s")

    @functools.partial(
        pl.kernel,
        mesh=mesh,
        compiler_params=pltpu.CompilerParams(needs_layout_passes=False),
        out_type=jax.ShapeDtypeStruct((B,), jnp.float32),
        scratch_types=[
            pltpu.VMEM((b_per_w,), jnp.int32),
            pltpu.VMEM((b_per_w,), jnp.int32),
            pltpu.VMEM((b_per_w,), jnp.int32),
            pltpu.VMEM((b_per_w,), jnp.int32),
            pltpu.VMEM((1, 128), jnp.float32),
            pltpu.VMEM((b_per_w, B), jnp.float32),
            pltpu.VMEM((b_per_w, B), jnp.float32),
            pltpu.VMEM((b_per_w * L,), jnp.float32),
            pltpu.VMEM((b_per_w * L,), jnp.float32),
            pltpu.VMEM((b_per_w,), jnp.float32),
            pltpu.SemaphoreType.DMA,
            pltpu.SemaphoreType.DMA,
        ],
    )
    def gather_k(pv_hbm, at_hbm, pg_hbm, tg_hbm, cp_hbm, out_hbm,
                 tg_v, cp_v, trow_v, crow_v, pgt, vrows, arows,
                 flatp, flata, out_v, sem_p, sem_a):
        wid = lax.axis_index("s") * NC + lax.axis_index("c")
        base = wid * b_per_w
        bcol = pl.multiple_of((wid // 4) * 128, 128)
        boff = (wid % 4) * b_per_w
        pltpu.sync_copy(tg_hbm.at[pl.ds(base, b_per_w)], tg_v)
        pltpu.sync_copy(cp_hbm.at[pl.ds(base, b_per_w)], cp_v)
        pltpu.sync_copy(pg_hbm.at[:, pl.ds(bcol, 128)], pgt)

        for j in range(n_chunks):
            sl = pl.ds(j * L, L)
            trow_v[sl] = jnp.minimum(jnp.maximum(tg_v[sl], 0), V - 1)
            crow_v[sl] = jnp.minimum(jnp.maximum(cp_v[sl], 0), S - 1)

        cp_p = pltpu.async_copy(pv_hbm.at[trow_v], vrows, sem_p)
        cp_a = pltpu.async_copy(at_hbm.at[crow_v], arows, sem_a)
        cp_p.wait()
        cp_a.wait()

        for i in range(b_per_w):
            run0 = pl.multiple_of(base + (i & ~15), L)
            flatp[pl.ds(i * L, L)] = vrows[i, pl.ds(run0, L)]
            flata[pl.ds(i * L, L)] = arows[i, pl.ds(run0, L)]

        iot = lax.iota(jnp.int32, L)
        for j in range(n_chunks):
            sl = pl.ds(j * L, L)
            idx = j * (L * L) + iot * (L + 1)
            pv_sel = plsc.load_gather(flatp, [idx])
            at_sel = plsc.load_gather(flata, [idx])
            g = tg_v[sl] < V
            pgv = pgt[0, pl.ds(pl.multiple_of(boff + j * L, L), L)]
            out_v[sl] = jnp.where(g, pgv * pv_sel + EPS,
                                  (1.0 - pgv) * at_sel + EPS)

        pltpu.sync_copy(out_v, out_hbm.at[pl.ds(base, b_per_w)])

    arg = gather_k(pvT, atT, pgT, tg, cp)

    def loss_body(x_ref, o_ref):
        o_ref[0, 0] = -jnp.sum(jnp.log(x_ref[...])) * (1.0 / B)

    loss = pl.pallas_call(
        loss_body,
        out_shape=jax.ShapeDtypeStruct((1, 1), jnp.float32),
        out_specs=pl.BlockSpec(memory_space=pltpu.SMEM),
    )(arg.reshape(B // 128, 128))
    return loss[0, 0]


# speedup vs baseline: 1.1887x; 1.1887x over previous
import functools

import jax
import jax.numpy as jnp
from jax import lax
from jax.experimental import pallas as pl
from jax.experimental.pallas import tpu as pltpu
from jax.experimental.pallas import tpu_sc as plsc

EPS = 1e-12
L = 16


def _tile_linear_1d(x):
    b, n = x.shape
    return x.reshape(b // 128, 128, n // 8, 8).transpose(2, 0, 3, 1).reshape(b * n)


def kernel(P_vocab, attn_dist, p_gen, target_idx, copy_idx):
    B, V = P_vocab.shape
    S = attn_dist.shape[1]

    pv1 = _tile_linear_1d(P_vocab)
    at1 = _tile_linear_1d(attn_dist)
    pg1 = p_gen.reshape(B)
    tg = target_idx.astype(jnp.int32)
    cp = copy_idx.astype(jnp.int32)

    info = plsc.get_sparse_core_info()
    NC, NS = info.num_cores, info.num_subcores
    NW = NC * NS
    b_per_w = B // NW
    n_chunks = b_per_w // L

    mesh = plsc.VectorSubcoreMesh(core_axis_name="c", subcore_axis_name="s")

    @functools.partial(
        pl.kernel,
        mesh=mesh,
        compiler_params=pltpu.CompilerParams(needs_layout_passes=False),
        out_type=jax.ShapeDtypeStruct((B,), jnp.float32),
        scratch_types=[
            pltpu.VMEM((b_per_w,), jnp.int32),
            pltpu.VMEM((b_per_w,), jnp.int32),
            pltpu.VMEM((b_per_w,), jnp.float32),
            pltpu.VMEM((b_per_w,), jnp.int32),
            pltpu.VMEM((b_per_w,), jnp.int32),
            pltpu.VMEM((b_per_w,), jnp.float32),
            pltpu.VMEM((b_per_w,), jnp.float32),
            pltpu.VMEM((b_per_w,), jnp.float32),
            pltpu.SemaphoreType.DMA,
            pltpu.SemaphoreType.DMA,
        ],
    )
    def gather_k(pv_hbm, at_hbm, pg_hbm, tg_hbm, cp_hbm, out_hbm,
                 tg_v, cp_v, pg_v, pidx_v, aidx_v, psel_v, asel_v,
                 out_v, sem_p, sem_a):
        wid = lax.axis_index("s") * NC + lax.axis_index("c")
        base = wid * b_per_w
        pltpu.sync_copy(tg_hbm.at[pl.ds(base, b_per_w)], tg_v)
        pltpu.sync_copy(cp_hbm.at[pl.ds(base, b_per_w)], cp_v)
        pltpu.sync_copy(pg_hbm.at[pl.ds(base, b_per_w)], pg_v)

        iot = lax.iota(jnp.int32, L)
        for j in range(n_chunks):
            sl = pl.ds(j * L, L)
            bvec = base + j * L + iot
            t = jnp.minimum(jnp.maximum(tg_v[sl], 0), V - 1)
            c = jnp.minimum(jnp.maximum(cp_v[sl], 0), S - 1)
            bpart = (lax.shift_right_logical(bvec, 7) * 1024
                     + (bvec & 127))
            pidx_v[sl] = (lax.shift_right_logical(t, 3) * 8192
                          + (t & 7) * 128 + bpart)
            aidx_v[sl] = (lax.shift_right_logical(c, 3) * 8192
                          + (c & 7) * 128 + bpart)

        cp_p = pltpu.async_copy(pv_hbm.at[pidx_v], psel_v, sem_p)
        cp_a = pltpu.async_copy(at_hbm.at[aidx_v], asel_v, sem_a)
        cp_p.wait()
        cp_a.wait()

        for j in range(n_chunks):
            sl = pl.ds(j * L, L)
            g = tg_v[sl] < V
            pgv = pg_v[sl]
            out_v[sl] = jnp.where(g, pgv * psel_v[sl] + EPS,
                                  (1.0 - pgv) * asel_v[sl] + EPS)

        pltpu.sync_copy(out_v, out_hbm.at[pl.ds(base, b_per_w)])

    arg = gather_k(pv1, at1, pg1, tg, cp)

    def loss_body(x_ref, o_ref):
        o_ref[0, 0] = -jnp.sum(jnp.log(x_ref[...])) * (1.0 / B)

    loss = pl.pallas_call(
        loss_body,
        out_shape=jax.ShapeDtypeStruct((1, 1), jnp.float32),
        out_specs=pl.BlockSpec(memory_space=pltpu.SMEM),
    )(arg.reshape(B // 128, 128))
    return loss[0, 0]


# speedup vs baseline: 1.2823x; 1.0788x over previous
import functools

import jax
import jax.numpy as jnp
from jax import lax
from jax.experimental import pallas as pl
from jax.experimental.pallas import tpu as pltpu
from jax.experimental.pallas import tpu_sc as plsc

EPS = 1e-12
L = 16


def _tile_linear_1d(x):
    b, n = x.shape
    return x.reshape(b // 128, 128, n // 8, 8).transpose(2, 0, 3, 1).reshape(b * n)


def kernel(P_vocab, attn_dist, p_gen, target_idx, copy_idx):
    B, V = P_vocab.shape
    S = attn_dist.shape[1]

    pv1 = _tile_linear_1d(P_vocab)
    at1 = _tile_linear_1d(attn_dist)
    pg1 = p_gen.reshape(B)
    tg = target_idx.astype(jnp.int32)
    cp = copy_idx.astype(jnp.int32)

    info = plsc.get_sparse_core_info()
    NC, NS = info.num_cores, info.num_subcores
    NW = NC * NS
    b_per_w = B // NW
    n_chunks = b_per_w // L

    mesh = plsc.VectorSubcoreMesh(core_axis_name="c", subcore_axis_name="s", num_cores=1)

    @functools.partial(
        pl.kernel,
        mesh=mesh,
        compiler_params=pltpu.CompilerParams(needs_layout_passes=False),
        out_type=jax.ShapeDtypeStruct((B,), jnp.float32),
        scratch_types=[
            pltpu.VMEM((b_per_w,), jnp.int32),
            pltpu.VMEM((b_per_w,), jnp.int32),
            pltpu.VMEM((b_per_w,), jnp.float32),
            pltpu.VMEM((b_per_w,), jnp.int32),
            pltpu.VMEM((b_per_w,), jnp.int32),
            pltpu.VMEM((b_per_w,), jnp.float32),
            pltpu.VMEM((b_per_w,), jnp.float32),
            pltpu.VMEM((b_per_w,), jnp.float32),
            pltpu.SemaphoreType.DMA,
            pltpu.SemaphoreType.DMA,
        ],
    )
    def gather_k(pv_hbm, at_hbm, pg_hbm, tg_hbm, cp_hbm, out_hbm,
                 tg_v, cp_v, pg_v, pidx_v, aidx_v, psel_v, asel_v,
                 out_v, sem_p, sem_a):
        wid = lax.axis_index("s") * NC + lax.axis_index("c")
        base = wid * b_per_w
        pltpu.sync_copy(tg_hbm.at[pl.ds(base, b_per_w)], tg_v)
        pltpu.sync_copy(cp_hbm.at[pl.ds(base, b_per_w)], cp_v)
        pltpu.sync_copy(pg_hbm.at[pl.ds(base, b_per_w)], pg_v)

        iot = lax.iota(jnp.int32, L)
        for j in range(n_chunks):
            sl = pl.ds(j * L, L)
            bvec = base + j * L + iot
            t = jnp.minimum(jnp.maximum(tg_v[sl], 0), V - 1)
            c = jnp.minimum(jnp.maximum(cp_v[sl], 0), S - 1)
            bpart = (lax.shift_right_logical(bvec, 7) * 1024
                     + (bvec & 127))
            pidx_v[sl] = (lax.shift_right_logical(t, 3) * 8192
                          + (t & 7) * 128 + bpart)
            aidx_v[sl] = (lax.shift_right_logical(c, 3) * 8192
                          + (c & 7) * 128 + bpart)

        cp_p = pltpu.async_copy(pv_hbm.at[pidx_v], psel_v, sem_p)
        cp_a = pltpu.async_copy(at_hbm.at[aidx_v], asel_v, sem_a)
        cp_p.wait()
        cp_a.wait()

        for j in range(n_chunks):
            sl = pl.ds(j * L, L)
            g = tg_v[sl] < V
            pgv = pg_v[sl]
            out_v[sl] = jnp.where(g, pgv * psel_v[sl] + EPS,
                                  (1.0 - pgv) * asel_v[sl] + EPS)

        pltpu.sync_copy(out_v, out_hbm.at[pl.ds(base, b_per_w)])

    arg = gather_k(pv1, at1, pg1, tg, cp)

    def loss_body(x_ref, o_ref):
        o_ref[0, 0] = -jnp.sum(jnp.log(x_ref[...])) * (1.0 / B)

    loss = pl.pallas_call(
        loss_body,
        out_shape=jax.ShapeDtypeStruct((1, 1), jnp.float32),
        out_specs=pl.BlockSpec(memory_space=pltpu.SMEM),
    )(arg.reshape(B // 128, 128))
    return loss[0, 0]
